# Initial kernel scaffold; baseline (speedup 1.0000x reference)
import functools

import jax
import jax.numpy as jnp
from jax import lax
from jax.experimental import pallas as pl
from jax.experimental.pallas import tpu as pltpu
from jax.experimental.pallas import tpu_sc as plsc

N_V = 10000
F = 320000
NUM_EMB = 100
DIM = 128

NC = 2
NS = 16
NW = NC * NS
E_PER_TILE = F // NW

KCOLS = 128
KROWS = (E_PER_TILE + KCOLS - 1) // KCOLS
E_PAD = KROWS * KCOLS

CNT = N_V * NUM_EMB
DST_PAD = 10240
CNT_PAD = DST_PAD * NUM_EMB
CHUNK = CNT_PAD // NS
ZBUF = 8000

_mesh = plsc.VectorSubcoreMesh(core_axis_name="c", subcore_axis_name="s")


@functools.partial(
    pl.kernel,
    out_type=jax.ShapeDtypeStruct((NC, CNT_PAD), jnp.float32),
    mesh=_mesh,
    scratch_types=[
        pltpu.VMEM((N_V,), jnp.int32),
        pltpu.VMEM((E_PAD,), jnp.int32),
        pltpu.VMEM((E_PAD,), jnp.int32),
        pltpu.VMEM((KROWS, KCOLS), jnp.int32),
        pltpu.VMEM((KROWS, KCOLS), jnp.float32),
        pltpu.VMEM((ZBUF,), jnp.float32),
        pltpu.VMEM_SHARED((CNT_PAD,), jnp.float32),
        pltpu.SemaphoreType.DMA,
    ],
)
def _hist_kernel(face_hbm, vx_hbm, out_hbm,
                 vxv, srcv, dstv, keys, payload, zbuf, hist, sem):
    cid = lax.axis_index("c")
    sid = lax.axis_index("s")
    wid = sid * NC + cid

    zeros16f = jnp.zeros((16,), jnp.float32)
    ones16f = jnp.ones((16,), jnp.float32)
    zeros16i = jnp.zeros((16,), jnp.int32)
    iota16 = lax.iota(jnp.int32, 16)

    def _zero_zbuf(i, c):
        zbuf[pl.ds(i * 16, 16)] = zeros16f
        return c
    lax.fori_loop(0, ZBUF // 16, _zero_zbuf, 0)
    for j in range(CHUNK // ZBUF):
        pltpu.sync_copy(zbuf, hist.at[pl.ds(sid * CHUNK + j * ZBUF, ZBUF)])

    base = wid * E_PER_TILE
    pltpu.sync_copy(vx_hbm, vxv)
    pltpu.sync_copy(face_hbm.at[0, pl.ds(base, E_PER_TILE)],
                    srcv.at[pl.ds(0, E_PER_TILE)])
    pltpu.sync_copy(face_hbm.at[1, pl.ds(base, E_PER_TILE)],
                    dstv.at[pl.ds(0, E_PER_TILE)])
    for k in range((E_PAD - E_PER_TILE) // 16):
        srcv[pl.ds(E_PER_TILE + k * 16, 16)] = zeros16i
        dstv[pl.ds(E_PER_TILE + k * 16, 16)] = zeros16i

    def _build(j, c):
        for g in range(KCOLS // 16):
            off = j * KCOLS + g * 16
            s16 = srcv[pl.ds(off, 16)]
            d16 = dstv[pl.ds(off, 16)]
            e16 = plsc.load_gather(vxv, [s16])
            keys[j, pl.ds(g * 16, 16)] = d16 * NUM_EMB + e16
            valid = (off + iota16) < E_PER_TILE
            payload[j, pl.ds(g * 16, 16)] = jnp.where(valid, ones16f, zeros16f)
        return c
    lax.fori_loop(0, KROWS, _build, 0)

    plsc.subcore_barrier()

    def _scatter(j, c):
        pltpu.sync_copy(payload.at[j], hist.at[keys.at[j]], add=True)
        return c
    lax.fori_loop(0, KROWS, _scatter, 0)

    plsc.subcore_barrier()

    pltpu.sync_copy(hist.at[pl.ds(sid * CHUNK, CHUNK)],
                    out_hbm.at[cid, pl.ds(sid * CHUNK, CHUNK)])


_BLK = 1000


def _tc_body(vx_in_ref, cnt_ref, tab_ref, vx_out_ref, ex_ref):
    tab = tab_ref[...]
    cnt = cnt_ref[0] + cnt_ref[1]
    ex_ref[...] = lax.dot_general(
        cnt, tab, (((1,), (0,)), ((), ())),
        precision=lax.Precision.HIGHEST, preferred_element_type=jnp.float32)
    ids = vx_in_ref[...]
    onehot = (ids == lax.broadcasted_iota(jnp.int32, (_BLK, NUM_EMB), 1)
              ).astype(jnp.float32)
    vx_out_ref[...] = lax.dot_general(
        onehot, tab, (((1,), (0,)), ((), ())),
        precision=lax.Precision.HIGHEST, preferred_element_type=jnp.float32)


_tc_call = pl.pallas_call(
    _tc_body,
    grid=(N_V // _BLK,),
    in_specs=[
        pl.BlockSpec((_BLK, 1), lambda i: (i, 0)),
        pl.BlockSpec((NC, _BLK, NUM_EMB), lambda i: (0, i, 0)),
        pl.BlockSpec((NUM_EMB, DIM), lambda i: (0, 0)),
    ],
    out_specs=[
        pl.BlockSpec((_BLK, DIM), lambda i: (i, 0)),
        pl.BlockSpec((_BLK, DIM), lambda i: (i, 0)),
    ],
    out_shape=[
        jax.ShapeDtypeStruct((N_V, DIM), jnp.float32),
        jax.ShapeDtypeStruct((N_V, DIM), jnp.float32),
    ],
)


def kernel(v_x, face_index, v_embed_table):
    counts = _hist_kernel(face_index, jnp.squeeze(v_x, axis=-1))
    counts = counts.reshape(NC, DST_PAD, NUM_EMB)
    vx, ex = _tc_call(v_x, counts, v_embed_table)
    return (vx, ex)


# speedup vs baseline: 17.3449x; 17.3449x over previous
import functools

import jax
import jax.numpy as jnp
from jax import lax
from jax.experimental import pallas as pl
from jax.experimental.pallas import tpu as pltpu
from jax.experimental.pallas import tpu_sc as plsc

N_V = 10000
F = 320000
NUM_EMB = 100
DIM = 128

NC = 2
NS = 16
NW = NC * NS
E_PER_TILE = F // NW

KCOLS = 128
KROWS = (E_PER_TILE + KCOLS - 1) // KCOLS
E_PAD = KROWS * KCOLS

CNT = N_V * NUM_EMB
DST_PAD = 10240
CNT_PAD = DST_PAD * NUM_EMB
CHUNK = CNT_PAD // NS
ZBUF = 8000

_mesh = plsc.VectorSubcoreMesh(core_axis_name="c", subcore_axis_name="s")


@functools.partial(
    pl.kernel,
    out_type=jax.ShapeDtypeStruct((NC, CNT_PAD), jnp.float32),
    mesh=_mesh,
    compiler_params=pltpu.CompilerParams(needs_layout_passes=False),
    scratch_types=[
        pltpu.VMEM((N_V,), jnp.int32),
        pltpu.VMEM((E_PAD,), jnp.int32),
        pltpu.VMEM((E_PAD,), jnp.int32),
        pltpu.VMEM((KROWS, KCOLS), jnp.int32),
        pltpu.VMEM((KROWS, KCOLS), jnp.float32),
        pltpu.VMEM((ZBUF,), jnp.float32),
        pltpu.VMEM_SHARED((CNT_PAD,), jnp.float32),
        pltpu.SemaphoreType.DMA,
    ],
)
def _hist_kernel(src_hbm, dst_hbm, vx_hbm, out_hbm,
                 vxv, srcv, dstv, keys, payload, zbuf, hist, sem):
    cid = lax.axis_index("c")
    sid = lax.axis_index("s")
    wid = sid * NC + cid

    zeros16f = jnp.zeros((16,), jnp.float32)
    ones16f = jnp.ones((16,), jnp.float32)
    zeros16i = jnp.zeros((16,), jnp.int32)
    iota16 = lax.iota(jnp.int32, 16)

    def _zero_zbuf(i, c):
        zbuf[pl.ds(i * 16, 16)] = zeros16f
        return c
    lax.fori_loop(0, ZBUF // 16, _zero_zbuf, 0)
    for j in range(CHUNK // ZBUF):
        pltpu.sync_copy(zbuf, hist.at[pl.ds(sid * CHUNK + j * ZBUF, ZBUF)])

    base = wid * E_PER_TILE
    pltpu.sync_copy(vx_hbm, vxv)
    pltpu.sync_copy(src_hbm.at[pl.ds(base, E_PER_TILE)],
                    srcv.at[pl.ds(0, E_PER_TILE)])
    pltpu.sync_copy(dst_hbm.at[pl.ds(base, E_PER_TILE)],
                    dstv.at[pl.ds(0, E_PER_TILE)])
    for k in range((E_PAD - E_PER_TILE) // 16):
        srcv[pl.ds(E_PER_TILE + k * 16, 16)] = zeros16i
        dstv[pl.ds(E_PER_TILE + k * 16, 16)] = zeros16i

    def _build(j, c):
        for g in range(KCOLS // 16):
            off = j * KCOLS + g * 16
            s16 = srcv[pl.ds(off, 16)]
            d16 = dstv[pl.ds(off, 16)]
            e16 = plsc.load_gather(vxv, [s16])
            keys[j, pl.ds(g * 16, 16)] = d16 * NUM_EMB + e16
            valid = (off + iota16) < E_PER_TILE
            payload[j, pl.ds(g * 16, 16)] = jnp.where(valid, ones16f, zeros16f)
        return c
    lax.fori_loop(0, KROWS, _build, 0)

    plsc.subcore_barrier()

    def _scatter(j, c):
        pltpu.sync_copy(payload.at[j], hist.at[keys.at[j]], add=True)
        return c
    lax.fori_loop(0, KROWS, _scatter, 0)

    plsc.subcore_barrier()

    pltpu.sync_copy(hist.at[pl.ds(sid * CHUNK, CHUNK)],
                    out_hbm.at[cid, pl.ds(sid * CHUNK, CHUNK)])


_BLK = 1000


def _tc_body(vx_in_ref, cnt_ref, tab_ref, vx_out_ref, ex_ref):
    tab = tab_ref[...]
    cnt = cnt_ref[0] + cnt_ref[1]
    ex_ref[...] = lax.dot_general(
        cnt, tab, (((1,), (0,)), ((), ())),
        precision=lax.Precision.HIGHEST, preferred_element_type=jnp.float32)
    ids = vx_in_ref[...]
    onehot = (ids == lax.broadcasted_iota(jnp.int32, (_BLK, NUM_EMB), 1)
              ).astype(jnp.float32)
    vx_out_ref[...] = lax.dot_general(
        onehot, tab, (((1,), (0,)), ((), ())),
        precision=lax.Precision.HIGHEST, preferred_element_type=jnp.float32)


_tc_call = pl.pallas_call(
    _tc_body,
    grid=(N_V // _BLK,),
    in_specs=[
        pl.BlockSpec((_BLK, 1), lambda i: (i, 0)),
        pl.BlockSpec((NC, _BLK, NUM_EMB), lambda i: (0, i, 0)),
        pl.BlockSpec((NUM_EMB, DIM), lambda i: (0, 0)),
    ],
    out_specs=[
        pl.BlockSpec((_BLK, DIM), lambda i: (i, 0)),
        pl.BlockSpec((_BLK, DIM), lambda i: (i, 0)),
    ],
    out_shape=[
        jax.ShapeDtypeStruct((N_V, DIM), jnp.float32),
        jax.ShapeDtypeStruct((N_V, DIM), jnp.float32),
    ],
)


def kernel(v_x, face_index, v_embed_table):
    counts = _hist_kernel(face_index[0], face_index[1],
                          jnp.squeeze(v_x, axis=-1))
    counts = counts.reshape(NC, DST_PAD, NUM_EMB)
    vx, ex = _tc_call(v_x, counts, v_embed_table)
    return (vx, ex)


# speedup vs baseline: 21.5944x; 1.2450x over previous
import functools

import jax
import jax.numpy as jnp
from jax import lax
from jax.experimental import pallas as pl
from jax.experimental.pallas import tpu as pltpu
from jax.experimental.pallas import tpu_sc as plsc

N_V = 10000
F = 320000
NUM_EMB = 100
DIM = 128

NC = 2
NS = 16
NW = NC * NS
E_PER_TILE = F // NW

KCOLS = 128
KROWS = (E_PER_TILE + KCOLS - 1) // KCOLS
E_PAD = KROWS * KCOLS

CNT = N_V * NUM_EMB
DST_PAD = 10240
CNT_PAD = DST_PAD * NUM_EMB
CHUNK = CNT_PAD // NS
ZBUF = 8000

_mesh = plsc.VectorSubcoreMesh(core_axis_name="c", subcore_axis_name="s")


@functools.partial(
    pl.kernel,
    out_type=jax.ShapeDtypeStruct((NC, CNT_PAD), jnp.float32),
    mesh=_mesh,
    compiler_params=pltpu.CompilerParams(needs_layout_passes=False),
    scratch_types=[
        pltpu.VMEM((N_V,), jnp.int32),
        pltpu.VMEM((E_PAD,), jnp.int32),
        pltpu.VMEM((E_PAD,), jnp.int32),
        pltpu.VMEM((KROWS, KCOLS), jnp.int32),
        pltpu.VMEM((KROWS, KCOLS), jnp.float32),
        pltpu.VMEM((ZBUF,), jnp.float32),
        pltpu.VMEM_SHARED((CNT_PAD,), jnp.float32),
        pltpu.SemaphoreType.DMA,
        pltpu.SemaphoreType.DMA,
    ],
)
def _hist_kernel(faces_hbm, vx_hbm, out_hbm,
                 vxv, srcv, dstv, keys, payload, zbuf, hist, sem_in, sem_sc):
    cid = lax.axis_index("c")
    sid = lax.axis_index("s")
    wid = sid * NC + cid

    zeros16f = jnp.zeros((16,), jnp.float32)
    ones16f = jnp.ones((16,), jnp.float32)
    zeros16i = jnp.zeros((16,), jnp.int32)
    iota16 = lax.iota(jnp.int32, 16)

    base = wid * E_PER_TILE
    cp_vx = pltpu.async_copy(vx_hbm, vxv, sem_in)
    cp_s = pltpu.async_copy(faces_hbm.at[pl.ds(base, E_PER_TILE)],
                            srcv.at[pl.ds(0, E_PER_TILE)], sem_in)
    cp_d = pltpu.async_copy(faces_hbm.at[pl.ds(F + base, E_PER_TILE)],
                            dstv.at[pl.ds(0, E_PER_TILE)], sem_in)

    def _zero_zbuf(i, c):
        zbuf[pl.ds(i * 16, 16)] = zeros16f
        return c
    lax.fori_loop(0, ZBUF // 16, _zero_zbuf, 0)
    for j in range(CHUNK // ZBUF):
        pltpu.sync_copy(zbuf, hist.at[pl.ds(sid * CHUNK + j * ZBUF, ZBUF)])

    plsc.subcore_barrier()

    cp_vx.wait()
    cp_s.wait()
    cp_d.wait()
    for k in range((E_PAD - E_PER_TILE) // 16):
        srcv[pl.ds(E_PER_TILE + k * 16, 16)] = zeros16i
        dstv[pl.ds(E_PER_TILE + k * 16, 16)] = zeros16i

    def _build(j, c):
        for g in range(KCOLS // 16):
            off = j * KCOLS + g * 16
            s16 = srcv[pl.ds(off, 16)]
            d16 = dstv[pl.ds(off, 16)]
            e16 = plsc.load_gather(vxv, [s16])
            keys[j, pl.ds(g * 16, 16)] = d16 * NUM_EMB + e16
            valid = (off + iota16) < E_PER_TILE
            payload[j, pl.ds(g * 16, 16)] = jnp.where(valid, ones16f, zeros16f)
        pltpu.async_copy(payload.at[j], hist.at[keys.at[j]], sem_sc, add=True)
        return c
    lax.fori_loop(0, KROWS, _build, 0)

    def _drain(j, c):
        pltpu.make_async_copy(payload.at[j], hist.at[keys.at[j]], sem_sc).wait()
        return c
    lax.fori_loop(0, KROWS, _drain, 0)

    plsc.subcore_barrier()

    pltpu.sync_copy(hist.at[pl.ds(sid * CHUNK, CHUNK)],
                    out_hbm.at[cid, pl.ds(sid * CHUNK, CHUNK)])


_BLK = 1000


def _tc_body(vx_in_ref, cnt_ref, tab_ref, vx_out_ref, ex_ref):
    tab = tab_ref[...]
    cnt = cnt_ref[0] + cnt_ref[1]
    ex_ref[...] = lax.dot_general(
        cnt, tab, (((1,), (0,)), ((), ())),
        precision=lax.Precision.HIGHEST, preferred_element_type=jnp.float32)
    ids = vx_in_ref[...]
    onehot = (ids == lax.broadcasted_iota(jnp.int32, (_BLK, NUM_EMB), 1)
              ).astype(jnp.float32)
    vx_out_ref[...] = lax.dot_general(
        onehot, tab, (((1,), (0,)), ((), ())),
        precision=lax.Precision.HIGHEST, preferred_element_type=jnp.float32)


_tc_call = pl.pallas_call(
    _tc_body,
    grid=(N_V // _BLK,),
    in_specs=[
        pl.BlockSpec((_BLK, 1), lambda i: (i, 0)),
        pl.BlockSpec((NC, _BLK, NUM_EMB), lambda i: (0, i, 0)),
        pl.BlockSpec((NUM_EMB, DIM), lambda i: (0, 0)),
    ],
    out_specs=[
        pl.BlockSpec((_BLK, DIM), lambda i: (i, 0)),
        pl.BlockSpec((_BLK, DIM), lambda i: (i, 0)),
    ],
    out_shape=[
        jax.ShapeDtypeStruct((N_V, DIM), jnp.float32),
        jax.ShapeDtypeStruct((N_V, DIM), jnp.float32),
    ],
)


def kernel(v_x, face_index, v_embed_table):
    counts = _hist_kernel(face_index.reshape(2 * F),
                          jnp.squeeze(v_x, axis=-1))
    counts = counts.reshape(NC, DST_PAD, NUM_EMB)
    vx, ex = _tc_call(v_x, counts, v_embed_table)
    return (vx, ex)


# speedup vs baseline: 32.6687x; 1.5128x over previous
import functools

import jax
import jax.numpy as jnp
from jax import lax
from jax.experimental import pallas as pl
from jax.experimental.pallas import tpu as pltpu
from jax.experimental.pallas import tpu_sc as plsc

N_V = 10000
F = 320000
NUM_EMB = 100
DIM = 128

NC = 2
NS = 16
NW = NC * NS

KCOLS = 128
SLAB = 10240
KROWS = SLAB // KCOLS
LAST_W = NW - 1
LAST_E = F - LAST_W * SLAB
LAST_ROWS = LAST_E // KCOLS

NBINS = 128
DST_PAD = 10240
CNT_PAD = DST_PAD * NBINS
CHUNK = CNT_PAD // NS
ZBUF = 8192

_mesh = plsc.VectorSubcoreMesh(core_axis_name="c", subcore_axis_name="s")


@functools.partial(
    pl.kernel,
    out_type=jax.ShapeDtypeStruct((NC * CNT_PAD,), jnp.float32),
    mesh=_mesh,
    compiler_params=pltpu.CompilerParams(needs_layout_passes=False),
    scratch_types=[
        pltpu.VMEM((N_V,), jnp.int32),
        pltpu.VMEM((2, SLAB), jnp.int32),
        pltpu.VMEM((KROWS, KCOLS), jnp.int32),
        pltpu.VMEM((KCOLS,), jnp.float32),
        pltpu.VMEM((ZBUF,), jnp.float32),
        pltpu.VMEM_SHARED((CNT_PAD,), jnp.float32),
        pltpu.SemaphoreType.DMA,
        pltpu.SemaphoreType.DMA,
    ],
)
def _hist_kernel(faces_hbm, vx_hbm, out_hbm,
                 vxv, slab, keys, ones_row, zbuf, hist, sem_in, sem_sc):
    cid = lax.axis_index("c")
    sid = lax.axis_index("s")
    wid = sid * NC + cid

    zeros16f = jnp.zeros((16,), jnp.float32)
    ones16f = jnp.ones((16,), jnp.float32)

    cp_vx = pltpu.async_copy(vx_hbm, vxv, sem_in)

    @pl.when(wid < LAST_W)
    def _load_full():
        pltpu.sync_copy(faces_hbm.at[:, pl.ds(wid * SLAB, SLAB)], slab)

    @pl.when(wid == LAST_W)
    def _load_rem():
        pltpu.sync_copy(faces_hbm.at[:, pl.ds(LAST_W * SLAB, LAST_E)],
                        slab.at[:, pl.ds(0, LAST_E)])

    for g in range(KCOLS // 16):
        ones_row[pl.ds(g * 16, 16)] = ones16f

    def _zero_zbuf(i, c):
        zbuf[pl.ds(i * 16, 16)] = zeros16f
        return c
    lax.fori_loop(0, ZBUF // 16, _zero_zbuf, 0)
    for j in range(CHUNK // ZBUF):
        pltpu.sync_copy(zbuf, hist.at[pl.ds(sid * CHUNK + j * ZBUF, ZBUF)])

    plsc.subcore_barrier()
    cp_vx.wait()

    nrows = jnp.where(wid == LAST_W, LAST_ROWS, KROWS)

    def _build(j, c):
        for g in range(KCOLS // 16):
            off = j * KCOLS + g * 16
            s16 = slab[0, pl.ds(off, 16)]
            d16 = slab[1, pl.ds(off, 16)]
            e16 = plsc.load_gather(vxv, [s16])
            keys[j, pl.ds(g * 16, 16)] = d16 * NBINS + e16
        pltpu.async_copy(ones_row, hist.at[keys.at[j]], sem_sc, add=True)
        return c
    lax.fori_loop(0, nrows, _build, 0)

    def _drain(j, c):
        pltpu.make_async_copy(ones_row, hist.at[keys.at[j]], sem_sc).wait()
        return c
    lax.fori_loop(0, nrows, _drain, 0)

    plsc.subcore_barrier()

    pltpu.sync_copy(hist.at[pl.ds(sid * CHUNK, CHUNK)],
                    out_hbm.at[pl.ds(cid * CNT_PAD + sid * CHUNK, CHUNK)])


_BLK = 1000

_DOT_DIMS = (((1,), (0,)), ((), ()))


def _tc_body(vx_in_ref, cnt_ref, tab_ref, vx_out_ref, ex_ref):
    tab = tab_ref[...]
    tab_hi = tab.astype(jnp.bfloat16)
    tab_lo = (tab - tab_hi.astype(jnp.float32)).astype(jnp.bfloat16)
    cnt = cnt_ref[0] + cnt_ref[1]
    cnt_hi = cnt.astype(jnp.bfloat16)
    cnt_lo = (cnt - cnt_hi.astype(jnp.float32)).astype(jnp.bfloat16)
    ex_ref[...] = (
        lax.dot_general(cnt_hi, tab_hi, _DOT_DIMS,
                        preferred_element_type=jnp.float32)
        + lax.dot_general(cnt_hi, tab_lo, _DOT_DIMS,
                          preferred_element_type=jnp.float32)
        + lax.dot_general(cnt_lo, tab_hi, _DOT_DIMS,
                          preferred_element_type=jnp.float32))
    ids = vx_in_ref[...]
    onehot = (ids == lax.broadcasted_iota(jnp.int32, (_BLK, NBINS), 1)
              ).astype(jnp.bfloat16)
    vx_out_ref[...] = (
        lax.dot_general(onehot, tab_hi, _DOT_DIMS,
                        preferred_element_type=jnp.float32)
        + lax.dot_general(onehot, tab_lo, _DOT_DIMS,
                          preferred_element_type=jnp.float32))


_tc_call = pl.pallas_call(
    _tc_body,
    grid=(N_V // _BLK,),
    in_specs=[
        pl.BlockSpec((_BLK, 1), lambda i: (i, 0)),
        pl.BlockSpec((NC, _BLK, NBINS), lambda i: (0, i, 0)),
        pl.BlockSpec((NBINS, DIM), lambda i: (0, 0)),
    ],
    out_specs=[
        pl.BlockSpec((_BLK, DIM), lambda i: (i, 0)),
        pl.BlockSpec((_BLK, DIM), lambda i: (i, 0)),
    ],
    out_shape=[
        jax.ShapeDtypeStruct((N_V, DIM), jnp.float32),
        jax.ShapeDtypeStruct((N_V, DIM), jnp.float32),
    ],
)


def kernel(v_x, face_index, v_embed_table):
    counts = _hist_kernel(face_index, jnp.squeeze(v_x, axis=-1))
    counts = counts.reshape(NC, DST_PAD, NBINS)
    tab_pad = jnp.pad(v_embed_table, ((0, NBINS - NUM_EMB), (0, 0)))
    vx, ex = _tc_call(v_x, counts, tab_pad)
    return (vx, ex)
